# baseline (speedup 1.0000x reference)
import jax
import jax.numpy as jnp
from jax.experimental import pallas as pl

ENC_DIM = 256
CODEBOOK_SIZE = 8192
BN = 128


def _vq_kernel(t_ref, cb_ref, cnt_ref, out_ref):
    t = t_ref[...]
    cb = cb_ref[...]
    inv = 1.0 / cnt_ref[...]
    g = jax.lax.dot_general(
        t, cb, (((1,), (1,)), ((), ())),
        preferred_element_type=jnp.float32)
    tsq = jnp.sum(t * t, axis=1, keepdims=True)
    csq = jnp.sum(cb * cb, axis=1)[None, :]
    d2 = tsq + csq * (inv * inv) - 2.0 * g * inv
    d2 = jnp.maximum(d2, 0.0)
    minv = jnp.min(d2, axis=1, keepdims=True)
    iota = jax.lax.broadcasted_iota(jnp.int32, d2.shape, 1)
    label = jnp.min(jnp.where(d2 == minv, iota, CODEBOOK_SIZE),
                    axis=1, keepdims=True)
    out_ref[...] = (iota == label).astype(jnp.float32)


def kernel(target, codebook, counts):
    n, d = target.shape
    k = codebook.shape[0]
    cnt = counts.reshape(1, k)
    grid = (n // BN,)
    return pl.pallas_call(
        _vq_kernel,
        grid=grid,
        in_specs=[
            pl.BlockSpec((BN, d), lambda i: (i, 0)),
            pl.BlockSpec((k, d), lambda i: (0, 0)),
            pl.BlockSpec((1, k), lambda i: (0, 0)),
        ],
        out_specs=pl.BlockSpec((BN, k), lambda i: (i, 0)),
        out_shape=jax.ShapeDtypeStruct((n, k), jnp.float32),
    )(target, codebook, cnt)


# speedup vs baseline: 1.7535x; 1.7535x over previous
import jax
import jax.numpy as jnp
from jax.experimental import pallas as pl
from jax.experimental.pallas import tpu as pltpu

BN = 128


def _vq_kernel(t_ref, cb_ref, cnt_ref, out_ref, cb2_ref, a_ref):
    k = cb_ref.shape[0]

    @pl.when(pl.program_id(0) == 0)
    def _init():
        inv = 1.0 / cnt_ref[...]
        cb = cb_ref[...]
        cb2_ref[...] = cb * (-2.0 * inv)
        a_col = jnp.sum(cb * cb, axis=1, keepdims=True) * (inv * inv)
        a_ref[...] = a_col.reshape(1, k)

    t = t_ref[...]
    g = jax.lax.dot_general(
        t, cb2_ref[...], (((1,), (1,)), ((), ())),
        preferred_element_type=jnp.float32)
    s = g + a_ref[...]
    tsq = jnp.sum(t * t, axis=1, keepdims=True)
    m = jnp.min(s, axis=1, keepdims=True)
    mc = jnp.maximum(m, -tsq)
    iota = jax.lax.broadcasted_iota(jnp.int32, s.shape, 1)
    label = jnp.min(jnp.where(s <= mc, iota, k),
                    axis=1, keepdims=True)
    out_ref[...] = jnp.where(iota == label, 1.0, 0.0).astype(jnp.float32)


def kernel(target, codebook, counts):
    n, d = target.shape
    k = codebook.shape[0]
    cnt = counts.reshape(k, 1)
    return pl.pallas_call(
        _vq_kernel,
        grid=(n // BN,),
        in_specs=[
            pl.BlockSpec((BN, d), lambda i: (i, 0)),
            pl.BlockSpec((k, d), lambda i: (0, 0)),
            pl.BlockSpec((k, 1), lambda i: (0, 0)),
        ],
        out_specs=pl.BlockSpec((BN, k), lambda i: (i, 0)),
        out_shape=jax.ShapeDtypeStruct((n, k), jnp.float32),
        scratch_shapes=[
            pltpu.VMEM((k, d), jnp.float32),
            pltpu.VMEM((1, k), jnp.float32),
        ],
    )(target, codebook, cnt)
